# Initial kernel scaffold; baseline (speedup 1.0000x reference)
import jax
import jax.numpy as jnp
from jax.experimental import pallas as pl


def kernel(x, W1, g1, b1, W2, g2, b2, k):
    raise NotImplementedError("write your pallas kernel here")


# speedup vs baseline: 2.7456x; 2.7456x over previous
import functools

import jax
import jax.numpy as jnp
from jax import lax
from jax.experimental import pallas as pl

NHEAD = 8
K = 10
HIGH = lax.Precision.HIGHEST


def _leaky(v):
    return jnp.where(v >= 0, v, 0.2 * v)


def _stage1_body(xf_ref, xt_ref, w1_ref, offs_ref, idx_ref, y1_ref, st_ref,
                 *, T, N, dims):
    b = pl.program_id(0)
    i = pl.program_id(1)
    xf = xf_ref[0]
    xt = xt_ref[0]
    coln = jnp.sum(xf * xf, axis=1)
    rown = jnp.sum(xt * xt, axis=1)
    ip = lax.dot_general(xt.astype(jnp.bfloat16), xf.astype(jnp.bfloat16),
                         (((1,), (1,)), ((), ())),
                         preferred_element_type=jnp.float32)
    p = (2.0 * ip - coln[None, :]) - rown[:, None]
    iota = lax.broadcasted_iota(jnp.int32, (T, N), 1)
    cols = []
    for _ in range(K):
        mx = jnp.max(p, axis=1, keepdims=True)
        cand = jnp.min(jnp.where(p == mx, iota, N), axis=1, keepdims=True)
        cols.append(cand)
        p = jnp.where(iota == cand, -jnp.inf, p)
    idx = jnp.concatenate(cols, axis=1) + offs_ref[0, 0]
    idx_ref[0] = idx

    w1 = w1_ref[...]
    s = jnp.zeros((dims,), jnp.float32)
    ss = jnp.zeros((dims,), jnp.float32)
    for m in range(K):
        onehot = (iota == idx[:, m:m + 1]).astype(jnp.float32)
        fm = lax.dot_general(onehot, xf, (((1,), (0,)), ((), ())),
                             preferred_element_type=jnp.float32,
                             precision=HIGH)
        feat = jnp.concatenate([fm - xt, xt], axis=1)
        y1m = lax.dot_general(feat, w1, (((1,), (1,)), ((), ())),
                              preferred_element_type=jnp.float32,
                              precision=HIGH)
        y1_ref[0, :, m * dims:(m + 1) * dims] = y1m
        s = s + jnp.sum(y1m, axis=0)
        ss = ss + jnp.sum(y1m * y1m, axis=0)
    pad = 128 - dims
    row_s = jnp.pad(s.reshape(1, dims), ((0, 0), (0, pad)))
    row_ss = jnp.pad(ss.reshape(1, dims), ((0, 0), (0, pad)))
    upd = jnp.concatenate([row_s, row_ss, jnp.zeros((6, 128), jnp.float32)],
                          axis=0)

    @pl.when(jnp.logical_and(b == 0, i == 0))
    def _():
        st_ref[...] = jnp.zeros_like(st_ref)

    st_ref[...] = st_ref[...] + upd


def _stage2_body(y1_ref, st1_ref, g1_ref, b1_ref, w2_ref, xout_ref, y2_ref,
                 st2_ref, *, T, dims, cnt):
    b = pl.program_id(0)
    i = pl.program_id(1)
    s = st1_ref[0, :dims]
    ss = st1_ref[1, :dims]
    mu = s / cnt
    var = ss / cnt - mu * mu
    r = lax.rsqrt(var + 1e-5)
    g1 = g1_ref[0]
    b1 = b1_ref[0]
    y1 = y1_ref[0].reshape(T, K, dims)
    eij = _leaky((y1 - mu) * r * g1 + b1)
    xout_ref[0] = jnp.max(eij, axis=1)
    w2 = w2_ref[0]
    y2 = jnp.sum(eij * w2[None, None, :], axis=2)
    y2_ref[0] = y2
    s2 = jnp.sum(y2)
    ss2 = jnp.sum(y2 * y2)
    upd = jnp.concatenate([jnp.full((1, 128), s2, jnp.float32),
                           jnp.full((1, 128), ss2, jnp.float32),
                           jnp.zeros((6, 128), jnp.float32)], axis=0)

    @pl.when(jnp.logical_and(b == 0, i == 0))
    def _():
        st2_ref[...] = jnp.zeros_like(st2_ref)

    st2_ref[...] = st2_ref[...] + upd


def _stage3_body(y2_ref, idx_ref, st2_ref, g2_ref, b2_ref, soft_ref, mask_ref,
                 *, T, N, cnt):
    mu = st2_ref[0, 0] / cnt
    var = st2_ref[1, 0] / cnt - mu * mu
    r = lax.rsqrt(var + 1e-5)
    g2 = g2_ref[0, 0]
    b2 = b2_ref[0, 0]
    e2 = _leaky((y2_ref[0] - mu) * r * g2 + b2)
    m = jnp.maximum(1.0, jnp.max(e2, axis=1, keepdims=True))
    ev = jnp.exp(e2 - m)
    base = jnp.exp(1.0 - m)
    denom = (N - K) * base + jnp.sum(ev, axis=1, keepdims=True)
    const = base / denom
    iota = lax.broadcasted_iota(jnp.int32, (T, N), 1)
    idx = idx_ref[0]
    out = jnp.broadcast_to(const, (T, N))
    hit = jnp.zeros((T, N), jnp.bool_)
    for mm in range(K):
        eq = iota == idx[:, mm:mm + 1]
        out = jnp.where(eq, ev[:, mm:mm + 1] / denom, out)
        hit = jnp.logical_or(hit, eq)
    soft_ref[0] = out
    mask_ref[0] = jnp.logical_not(hit)


def kernel(x, W1, g1, b1, W2, g2, b2, k):
    S, B, D = x.shape
    head = D // NHEAD
    xT = jnp.transpose(
        jnp.transpose(x, (1, 0, 2)).reshape(S, B * NHEAD, head), (1, 0, 2))
    Bh, N, dims = xT.shape
    T = min(256, N)
    nT = N // T
    cnt1 = float(Bh * N * K)
    offs = (jnp.asarray(k, jnp.int32) - K).reshape(1, 1)

    idx, y1, st1 = pl.pallas_call(
        functools.partial(_stage1_body, T=T, N=N, dims=dims),
        grid=(Bh, nT),
        in_specs=[
            pl.BlockSpec((1, N, dims), lambda b, i: (b, 0, 0)),
            pl.BlockSpec((1, T, dims), lambda b, i: (b, i, 0)),
            pl.BlockSpec((dims, 2 * dims), lambda b, i: (0, 0)),
            pl.BlockSpec((1, 1), lambda b, i: (0, 0)),
        ],
        out_specs=[
            pl.BlockSpec((1, T, K), lambda b, i: (b, i, 0)),
            pl.BlockSpec((1, T, K * dims), lambda b, i: (b, i, 0)),
            pl.BlockSpec((8, 128), lambda b, i: (0, 0)),
        ],
        out_shape=[
            jax.ShapeDtypeStruct((Bh, N, K), jnp.int32),
            jax.ShapeDtypeStruct((Bh, N, K * dims), jnp.float32),
            jax.ShapeDtypeStruct((8, 128), jnp.float32),
        ],
    )(xT, xT, W1, offs)

    xout, y2, st2 = pl.pallas_call(
        functools.partial(_stage2_body, T=T, dims=dims, cnt=cnt1),
        grid=(Bh, nT),
        in_specs=[
            pl.BlockSpec((1, T, K * dims), lambda b, i: (b, i, 0)),
            pl.BlockSpec((8, 128), lambda b, i: (0, 0)),
            pl.BlockSpec((1, dims), lambda b, i: (0, 0)),
            pl.BlockSpec((1, dims), lambda b, i: (0, 0)),
            pl.BlockSpec((1, dims), lambda b, i: (0, 0)),
        ],
        out_specs=[
            pl.BlockSpec((1, T, dims), lambda b, i: (b, i, 0)),
            pl.BlockSpec((1, T, K), lambda b, i: (b, i, 0)),
            pl.BlockSpec((8, 128), lambda b, i: (0, 0)),
        ],
        out_shape=[
            jax.ShapeDtypeStruct((Bh, N, dims), jnp.float32),
            jax.ShapeDtypeStruct((Bh, N, K), jnp.float32),
            jax.ShapeDtypeStruct((8, 128), jnp.float32),
        ],
    )(y1, st1, g1.reshape(1, dims), b1.reshape(1, dims), W2.reshape(1, dims))

    soft, mask = pl.pallas_call(
        functools.partial(_stage3_body, T=T, N=N, cnt=cnt1),
        grid=(Bh, nT),
        in_specs=[
            pl.BlockSpec((1, T, K), lambda b, i: (b, i, 0)),
            pl.BlockSpec((1, T, K), lambda b, i: (b, i, 0)),
            pl.BlockSpec((8, 128), lambda b, i: (0, 0)),
            pl.BlockSpec((1, 1), lambda b, i: (0, 0)),
            pl.BlockSpec((1, 1), lambda b, i: (0, 0)),
        ],
        out_specs=[
            pl.BlockSpec((1, T, N), lambda b, i: (b, i, 0)),
            pl.BlockSpec((1, T, N), lambda b, i: (b, i, 0)),
        ],
        out_shape=[
            jax.ShapeDtypeStruct((Bh, N, N), jnp.float32),
            jax.ShapeDtypeStruct((Bh, N, N), jnp.bool_),
        ],
    )(y2, idx, st2, g2.reshape(1, 1), b2.reshape(1, 1))

    return (xout, soft, mask)


# speedup vs baseline: 8.5649x; 3.1195x over previous
import functools

import jax
import jax.numpy as jnp
from jax import lax
from jax.experimental import pallas as pl

NHEAD = 8
K = 10
HIGH = lax.Precision.HIGHEST


def _leaky(v):
    return jnp.where(v >= 0, v, 0.2 * v)


def _stage1_body(xf_ref, xt_ref, w1_ref, offs_ref, idx_ref, y1_ref, st_ref,
                 *, T, N, dims):
    b = pl.program_id(0)
    i = pl.program_id(1)
    xf = xf_ref[0]
    xt = xt_ref[0]
    coln = jnp.sum(xf * xf, axis=1)
    rown = jnp.sum(xt * xt, axis=1)
    ip = lax.dot_general(xt.astype(jnp.bfloat16), xf.astype(jnp.bfloat16),
                         (((1,), (1,)), ((), ())),
                         preferred_element_type=jnp.float32)
    p = (2.0 * ip - coln[None, :]) - rown[:, None]

    w1 = w1_ref[...]
    w1a = w1[:, :dims].astype(jnp.bfloat16)
    w1d = (w1[:, dims:] - w1[:, :dims]).astype(jnp.bfloat16)
    xw = lax.dot_general(xf.astype(jnp.bfloat16), w1a,
                         (((1,), (1,)), ((), ())),
                         preferred_element_type=jnp.float32)
    xw = xw.astype(jnp.bfloat16)
    z = lax.dot_general(xt.astype(jnp.bfloat16), w1d,
                        (((1,), (1,)), ((), ())),
                        preferred_element_type=jnp.float32)

    iota = lax.broadcasted_iota(jnp.int32, (T, N), 1)
    cols = []
    s = jnp.zeros((dims,), jnp.float32)
    ss = jnp.zeros((dims,), jnp.float32)
    for m in range(K):
        mx = jnp.max(p, axis=1, keepdims=True)
        cand = jnp.min(jnp.where(p == mx, iota, N), axis=1, keepdims=True)
        cols.append(cand)
        eqm = iota == cand
        p = jnp.where(eqm, -jnp.inf, p)
        onehot = jnp.where(eqm, 1.0, 0.0).astype(jnp.bfloat16)
        gm = lax.dot_general(onehot, xw, (((1,), (0,)), ((), ())),
                             preferred_element_type=jnp.float32)
        y1m = gm + z
        y1_ref[0, :, m * dims:(m + 1) * dims] = y1m
        s = s + jnp.sum(y1m, axis=0)
        ss = ss + jnp.sum(y1m * y1m, axis=0)
    idx_ref[0] = jnp.concatenate(cols, axis=1) + offs_ref[0, 0]
    pad = 128 - dims
    row_s = jnp.pad(s.reshape(1, dims), ((0, 0), (0, pad)))
    row_ss = jnp.pad(ss.reshape(1, dims), ((0, 0), (0, pad)))
    upd = jnp.concatenate([row_s, row_ss, jnp.zeros((6, 128), jnp.float32)],
                          axis=0)

    @pl.when(jnp.logical_and(b == 0, i == 0))
    def _():
        st_ref[...] = jnp.zeros_like(st_ref)

    st_ref[...] = st_ref[...] + upd


def _stage2_body(y1_ref, st1_ref, g1_ref, b1_ref, w2_ref, xout_ref, y2_ref,
                 st2_ref, *, T, dims, cnt):
    b = pl.program_id(0)
    i = pl.program_id(1)
    s = st1_ref[0, :dims]
    ss = st1_ref[1, :dims]
    mu = s / cnt
    var = ss / cnt - mu * mu
    r = lax.rsqrt(var + 1e-5)
    g1 = g1_ref[0]
    b1 = b1_ref[0]
    y1 = y1_ref[0].reshape(T, K, dims)
    eij = _leaky((y1 - mu) * r * g1 + b1)
    xout_ref[0] = jnp.max(eij, axis=1)
    w2 = w2_ref[0]
    y2 = jnp.sum(eij * w2[None, None, :], axis=2)
    y2_ref[0] = y2
    s2 = jnp.sum(y2)
    ss2 = jnp.sum(y2 * y2)
    upd = jnp.concatenate([jnp.full((1, 128), s2, jnp.float32),
                           jnp.full((1, 128), ss2, jnp.float32),
                           jnp.zeros((6, 128), jnp.float32)], axis=0)

    @pl.when(jnp.logical_and(b == 0, i == 0))
    def _():
        st2_ref[...] = jnp.zeros_like(st2_ref)

    st2_ref[...] = st2_ref[...] + upd


def _stage3_body(y2_ref, idx_ref, st2_ref, g2_ref, b2_ref, soft_ref, mask_ref,
                 *, T, N, cnt):
    mu = st2_ref[0, 0] / cnt
    var = st2_ref[1, 0] / cnt - mu * mu
    r = lax.rsqrt(var + 1e-5)
    g2 = g2_ref[0, 0]
    b2 = b2_ref[0, 0]
    e2 = _leaky((y2_ref[0] - mu) * r * g2 + b2)
    m = jnp.maximum(1.0, jnp.max(e2, axis=1, keepdims=True))
    ev = jnp.exp(e2 - m)
    base = jnp.exp(1.0 - m)
    denom = (N - K) * base + jnp.sum(ev, axis=1, keepdims=True)
    const = base / denom
    iota = lax.broadcasted_iota(jnp.int32, (T, N), 1)
    idx = idx_ref[0]
    out = jnp.broadcast_to(const, (T, N))
    hit = jnp.zeros((T, N), jnp.bool_)
    for mm in range(K):
        eq = iota == idx[:, mm:mm + 1]
        out = jnp.where(eq, ev[:, mm:mm + 1] / denom, out)
        hit = jnp.logical_or(hit, eq)
    soft_ref[0] = out
    mask_ref[0] = jnp.logical_not(hit)


def kernel(x, W1, g1, b1, W2, g2, b2, k):
    S, B, D = x.shape
    head = D // NHEAD
    xT = jnp.transpose(
        jnp.transpose(x, (1, 0, 2)).reshape(S, B * NHEAD, head), (1, 0, 2))
    Bh, N, dims = xT.shape
    T = min(256, N)
    nT = N // T
    cnt1 = float(Bh * N * K)
    offs = (jnp.asarray(k, jnp.int32) - K).reshape(1, 1)

    idx, y1, st1 = pl.pallas_call(
        functools.partial(_stage1_body, T=T, N=N, dims=dims),
        grid=(Bh, nT),
        in_specs=[
            pl.BlockSpec((1, N, dims), lambda b, i: (b, 0, 0)),
            pl.BlockSpec((1, T, dims), lambda b, i: (b, i, 0)),
            pl.BlockSpec((dims, 2 * dims), lambda b, i: (0, 0)),
            pl.BlockSpec((1, 1), lambda b, i: (0, 0)),
        ],
        out_specs=[
            pl.BlockSpec((1, T, K), lambda b, i: (b, i, 0)),
            pl.BlockSpec((1, T, K * dims), lambda b, i: (b, i, 0)),
            pl.BlockSpec((8, 128), lambda b, i: (0, 0)),
        ],
        out_shape=[
            jax.ShapeDtypeStruct((Bh, N, K), jnp.int32),
            jax.ShapeDtypeStruct((Bh, N, K * dims), jnp.float32),
            jax.ShapeDtypeStruct((8, 128), jnp.float32),
        ],
    )(xT, xT, W1, offs)

    xout, y2, st2 = pl.pallas_call(
        functools.partial(_stage2_body, T=T, dims=dims, cnt=cnt1),
        grid=(Bh, nT),
        in_specs=[
            pl.BlockSpec((1, T, K * dims), lambda b, i: (b, i, 0)),
            pl.BlockSpec((8, 128), lambda b, i: (0, 0)),
            pl.BlockSpec((1, dims), lambda b, i: (0, 0)),
            pl.BlockSpec((1, dims), lambda b, i: (0, 0)),
            pl.BlockSpec((1, dims), lambda b, i: (0, 0)),
        ],
        out_specs=[
            pl.BlockSpec((1, T, dims), lambda b, i: (b, i, 0)),
            pl.BlockSpec((1, T, K), lambda b, i: (b, i, 0)),
            pl.BlockSpec((8, 128), lambda b, i: (0, 0)),
        ],
        out_shape=[
            jax.ShapeDtypeStruct((Bh, N, dims), jnp.float32),
            jax.ShapeDtypeStruct((Bh, N, K), jnp.float32),
            jax.ShapeDtypeStruct((8, 128), jnp.float32),
        ],
    )(y1, st1, g1.reshape(1, dims), b1.reshape(1, dims), W2.reshape(1, dims))

    soft, mask = pl.pallas_call(
        functools.partial(_stage3_body, T=T, N=N, cnt=cnt1),
        grid=(Bh, nT),
        in_specs=[
            pl.BlockSpec((1, T, K), lambda b, i: (b, i, 0)),
            pl.BlockSpec((1, T, K), lambda b, i: (b, i, 0)),
            pl.BlockSpec((8, 128), lambda b, i: (0, 0)),
            pl.BlockSpec((1, 1), lambda b, i: (0, 0)),
            pl.BlockSpec((1, 1), lambda b, i: (0, 0)),
        ],
        out_specs=[
            pl.BlockSpec((1, T, N), lambda b, i: (b, i, 0)),
            pl.BlockSpec((1, T, N), lambda b, i: (b, i, 0)),
        ],
        out_shape=[
            jax.ShapeDtypeStruct((Bh, N, N), jnp.float32),
            jax.ShapeDtypeStruct((Bh, N, N), jnp.bool_),
        ],
    )(y2, idx, st2, g2.reshape(1, 1), b2.reshape(1, 1))

    return (xout, soft, mask)
